# baseline (speedup 1.0000x reference)
import functools

import jax
import jax.numpy as jnp
from jax import lax
from jax.experimental import pallas as pl
from jax.experimental.pallas import tpu as pltpu
from jax.experimental.pallas import tpu_sc as plsc

B = 16384
D = 64
NW = 32
BPW = B // NW
CH = 128
NC = BPW // CH
NS = 8
RPS = CH // NS

_mesh = plsc.VectorSubcoreMesh(core_axis_name="c", subcore_axis_name="s")


@functools.partial(
    pl.kernel,
    out_type=(jax.ShapeDtypeStruct((B, D), jnp.float32),
              jax.ShapeDtypeStruct((B, D), jnp.float32)),
    mesh=_mesh,
    scratch_types=[
        pltpu.VMEM((BPW,), jnp.int32),
        pltpu.VMEM((BPW,), jnp.int32),
        pltpu.VMEM((2, CH, D), jnp.float32),
        pltpu.VMEM((2, CH, D), jnp.float32),
    ] + [pltpu.SemaphoreType.DMA] * (2 * NS),
)
def _sc_gather(uidx_hbm, iidx_hbm, utab_hbm, itab_hbm, uout_hbm, iout_hbm,
               uidx_v, iidx_v, ubuf, ibuf, *sems):
    semus = sems[:NS]
    semis = sems[NS:]
    wid = lax.axis_index("s") * _mesh.num_cores + lax.axis_index("c")
    base = wid * BPW
    pltpu.sync_copy(uidx_hbm.at[pl.ds(base, BPW)], uidx_v)
    pltpu.sync_copy(iidx_hbm.at[pl.ds(base, BPW)], iidx_v)

    def fire(ch, b):
        off = ch * CH

        def f(g, _):
            uvec = uidx_v[pl.ds(off + g * 16, 16)]
            ivec = iidx_v[pl.ds(off + g * 16, 16)]
            for l in range(16):
                ui = uvec[l]
                ii = ivec[l]
                pltpu.async_copy(utab_hbm.at[pl.ds(ui, 1)],
                                 ubuf.at[b, pl.ds(g * 16 + l, 1)],
                                 semus[l % NS])
                pltpu.async_copy(itab_hbm.at[pl.ds(ii, 1)],
                                 ibuf.at[b, pl.ds(g * 16 + l, 1)],
                                 semis[l % NS])
            return 0

        lax.fori_loop(0, CH // 16, f, 0)

    def drain_and_flush(ch, b):
        for s in range(NS):
            pltpu.make_async_copy(utab_hbm.at[pl.ds(0, RPS)],
                                  ubuf.at[b, pl.ds(0, RPS)], semus[s]).wait()
            pltpu.make_async_copy(itab_hbm.at[pl.ds(0, RPS)],
                                  ibuf.at[b, pl.ds(0, RPS)], semis[s]).wait()
        pltpu.sync_copy(ubuf.at[b], uout_hbm.at[pl.ds(base + ch * CH, CH)])
        pltpu.sync_copy(ibuf.at[b], iout_hbm.at[pl.ds(base + ch * CH, CH)])

    fire(0, 0)
    fire(1, 1)
    for ch in range(2, NC):
        drain_and_flush(ch - 2, (ch - 2) % 2)
        fire(ch, ch % 2)
    drain_and_flush(NC - 2, (NC - 2) % 2)
    drain_and_flush(NC - 1, (NC - 1) % 2)


BLK = 2048


def _mlp_body(u_ref, i_ref, w1u_ref, w1i_ref, b1_ref, w2_ref, b2_ref,
              w3_ref, b3_ref, o_ref):
    h = jnp.dot(u_ref[...], w1u_ref[...], preferred_element_type=jnp.float32)
    h = h + jnp.dot(i_ref[...], w1i_ref[...], preferred_element_type=jnp.float32)
    h = jnp.maximum(h + b1_ref[...], 0.0)
    h = jnp.dot(h, w2_ref[...], preferred_element_type=jnp.float32) + b2_ref[...]
    h = jnp.maximum(h, 0.0)
    z = jnp.dot(h, w3_ref[...], preferred_element_type=jnp.float32) + b3_ref[...]
    o_ref[...] = 1.0 / (1.0 + jnp.exp(-z))


def kernel(user_indices, item_indices, user_table, item_table,
           W1, b1, W2, b2, W3, b3):
    uidx = user_indices.astype(jnp.int32)
    iidx = item_indices.astype(jnp.int32)
    urows, irows = _sc_gather(uidx, iidx, user_table, item_table)

    w1u = W1[:, :D].T
    w1i = W1[:, D:].T
    w2 = W2.T
    w3 = W3.T

    return pl.pallas_call(
        _mlp_body,
        grid=(B // BLK,),
        in_specs=[
            pl.BlockSpec((BLK, D), lambda g: (g, 0)),
            pl.BlockSpec((BLK, D), lambda g: (g, 0)),
            pl.BlockSpec((D, 64), lambda g: (0, 0)),
            pl.BlockSpec((D, 64), lambda g: (0, 0)),
            pl.BlockSpec((1, 64), lambda g: (0, 0)),
            pl.BlockSpec((64, 32), lambda g: (0, 0)),
            pl.BlockSpec((1, 32), lambda g: (0, 0)),
            pl.BlockSpec((32, 1), lambda g: (0, 0)),
            pl.BlockSpec((1, 1), lambda g: (0, 0)),
        ],
        out_specs=pl.BlockSpec((BLK, 1), lambda g: (g, 0)),
        out_shape=jax.ShapeDtypeStruct((B, 1), jnp.float32),
    )(urows, irows, w1u, w1i, b1.reshape(1, 64), w2, b2.reshape(1, 32),
      w3, b3.reshape(1, 1))


# speedup vs baseline: 1.0820x; 1.0820x over previous
import functools

import jax
import jax.numpy as jnp
from jax import lax
from jax.experimental import pallas as pl
from jax.experimental.pallas import tpu as pltpu
from jax.experimental.pallas import tpu_sc as plsc

B = 16384
D = 64
NW = 32
BPW = B // NW
CH = 128
NC = BPW // CH

_mesh = plsc.VectorSubcoreMesh(core_axis_name="c", subcore_axis_name="s")


@functools.partial(
    pl.kernel,
    out_type=jax.ShapeDtypeStruct((B, D), jnp.float32),
    mesh=_mesh,
    scratch_types=[
        pltpu.VMEM((BPW,), jnp.int32),
        pltpu.VMEM((2, CH, D), jnp.float32),
        pltpu.SemaphoreType.DMA,
        pltpu.SemaphoreType.DMA,
    ],
)
def _sc_gather1(idx_hbm, tab_hbm, out_hbm, idx_v, buf, sem0, sem1):
    wid = lax.axis_index("s") * _mesh.num_cores + lax.axis_index("c")
    base = wid * BPW
    pltpu.sync_copy(idx_hbm.at[pl.ds(base, BPW)], idx_v)

    sems = (sem0, sem1)

    def fire(ch, b):
        off = ch * CH

        def f(g, _):
            vec = idx_v[pl.ds(off + g * 16, 16)]
            for l in range(16):
                pltpu.async_copy(tab_hbm.at[pl.ds(vec[l], 1)],
                                 buf.at[b, pl.ds(g * 16 + l, 1)], sems[b])
            return 0

        lax.fori_loop(0, CH // 16, f, 0)

    def drain_and_flush(ch, b):
        pltpu.make_async_copy(tab_hbm.at[pl.ds(0, CH)], buf.at[b],
                              sems[b]).wait()
        pltpu.sync_copy(buf.at[b], out_hbm.at[pl.ds(base + ch * CH, CH)])

    fire(0, 0)
    fire(1, 1)
    for ch in range(2, NC):
        drain_and_flush(ch - 2, (ch - 2) % 2)
        fire(ch, ch % 2)
    drain_and_flush(NC - 2, (NC - 2) % 2)
    drain_and_flush(NC - 1, (NC - 1) % 2)


BLK = 2048


def _mlp_body(u_ref, i_ref, w1u_ref, w1i_ref, b1_ref, w2_ref, b2_ref,
              w3_ref, b3_ref, o_ref):
    h = jnp.dot(u_ref[...], w1u_ref[...], preferred_element_type=jnp.float32)
    h = h + jnp.dot(i_ref[...], w1i_ref[...], preferred_element_type=jnp.float32)
    h = jnp.maximum(h + b1_ref[...], 0.0)
    h = jnp.dot(h, w2_ref[...], preferred_element_type=jnp.float32) + b2_ref[...]
    h = jnp.maximum(h, 0.0)
    z = jnp.dot(h, w3_ref[...], preferred_element_type=jnp.float32) + b3_ref[...]
    o_ref[...] = 1.0 / (1.0 + jnp.exp(-z))


def kernel(user_indices, item_indices, user_table, item_table,
           W1, b1, W2, b2, W3, b3):
    uidx = user_indices.astype(jnp.int32)
    iidx = item_indices.astype(jnp.int32)
    urows = _sc_gather1(uidx, user_table)
    irows = _sc_gather1(iidx, item_table)

    w1u = W1[:, :D].T
    w1i = W1[:, D:].T
    w2 = W2.T
    w3 = W3.T

    return pl.pallas_call(
        _mlp_body,
        grid=(B // BLK,),
        in_specs=[
            pl.BlockSpec((BLK, D), lambda g: (g, 0)),
            pl.BlockSpec((BLK, D), lambda g: (g, 0)),
            pl.BlockSpec((D, 64), lambda g: (0, 0)),
            pl.BlockSpec((D, 64), lambda g: (0, 0)),
            pl.BlockSpec((1, 64), lambda g: (0, 0)),
            pl.BlockSpec((64, 32), lambda g: (0, 0)),
            pl.BlockSpec((1, 32), lambda g: (0, 0)),
            pl.BlockSpec((32, 1), lambda g: (0, 0)),
            pl.BlockSpec((1, 1), lambda g: (0, 0)),
        ],
        out_specs=pl.BlockSpec((BLK, 1), lambda g: (g, 0)),
        out_shape=jax.ShapeDtypeStruct((B, 1), jnp.float32),
    )(urows, irows, w1u, w1i, b1.reshape(1, 64), w2, b2.reshape(1, 32),
      w3, b3.reshape(1, 1))
